# baseline (speedup 1.0000x reference)
import jax
import jax.numpy as jnp
from jax.experimental import pallas as pl

N_NODES = 100000
N_EDGES = 1600000
NUM_GRAPHS = 200
N = 20
T = 10
NUM_TIMEPTS = 50
H = 32

EBLK = 12800


def _mlp_block(feat_ref, w1_ref, b1_ref, w2_ref, b2_ref, out_ref):
    x = feat_ref[...]
    h1 = jax.nn.relu(
        jax.lax.dot_general(x, w1_ref[...], (((1,), (0,)), ((), ())),
                            preferred_element_type=jnp.float32) + b1_ref[...])
    out_ref[...] = jax.lax.dot_general(
        h1, w2_ref[...], (((1,), (0,)), ((), ())),
        preferred_element_type=jnp.float32) + b2_ref[...]


def _edge_mlp(feat, W1, b1, W2, b2):
    E, F = feat.shape
    grid = E // EBLK
    return pl.pallas_call(
        _mlp_block,
        grid=(grid,),
        in_specs=[
            pl.BlockSpec((EBLK, F), lambda i: (i, 0)),
            pl.BlockSpec((F, H), lambda i: (0, 0)),
            pl.BlockSpec((1, H), lambda i: (0, 0)),
            pl.BlockSpec((H, H), lambda i: (0, 0)),
            pl.BlockSpec((1, H), lambda i: (0, 0)),
        ],
        out_specs=pl.BlockSpec((EBLK, H), lambda i: (i, 0)),
        out_shape=jax.ShapeDtypeStruct((E, H), jnp.float32),
    )(feat, W1, b1.reshape(1, H), W2, b2.reshape(1, H))


def _point_layer(h, pos, src, dst, W1, b1, W2, b2):
    feat = jnp.concatenate([h[src], pos[src] - pos[dst]], axis=-1)
    F = feat.shape[1]
    Fpad = 40 if F == 35 else 8
    feat = jnp.pad(feat, ((0, 0), (0, Fpad - F)))
    W1p = jnp.pad(W1, ((0, Fpad - F), (0, 0)))
    m = _edge_mlp(feat, W1p, b1, W2, b2)
    agg = jax.ops.segment_max(m, dst, num_segments=N_NODES)
    return jnp.where(jnp.isneginf(agg), 0.0, agg)


def kernel(pos, edge_index, batch, pts_tid, pts_msk, pts_aux,
           W1a, b1a, W2a, b2a, W1b, b1b, W2b, b2b):
    src, dst = edge_index[0], edge_index[1]
    h = jax.nn.relu(_point_layer(pos, pos, src, dst, W1a, b1a, W2a, b2a))
    h = jax.nn.relu(_point_layer(h, pos, src, dst, W1b, b1b, W2b, b2b))
    enc_g = jax.ops.segment_max(h, batch, num_segments=NUM_GRAPHS)
    enc_g = jnp.where(jnp.isneginf(enc_g), 0.0, enc_g)
    enc = enc_g.reshape(N, T, H)
    parts_inp_obs = jnp.zeros((N, NUM_TIMEPTS, H), jnp.float32).at[:, :T].set(enc)
    parts_inp_msk = jnp.zeros((N, NUM_TIMEPTS, H), jnp.float32).at[:, :T].set(1.0)
    parts_inp_tps = jnp.zeros((N, NUM_TIMEPTS), jnp.float32).at[:, :T].set(
        pts_tid.astype(jnp.float32) / NUM_TIMEPTS)
    evd_obs = jnp.zeros((N, NUM_TIMEPTS, H), jnp.float32).at[
        jnp.arange(N)[:, None], pts_tid, :].set(enc)
    evd_msk = jnp.broadcast_to(pts_msk, (N, NUM_TIMEPTS, H))
    return (parts_inp_obs, parts_inp_msk, parts_inp_tps, evd_obs, evd_msk, pts_aux)
